# Initial kernel scaffold; baseline (speedup 1.0000x reference)
import jax
import jax.numpy as jnp
from jax.experimental import pallas as pl


def kernel(embedding, embedding_collection, labels_int):
    raise NotImplementedError("write your pallas kernel here")


# speedup vs baseline: 2.4664x; 2.4664x over previous
import jax
import jax.numpy as jnp
from jax.experimental import pallas as pl

N = 100000
D = 128
BLK = 2000
GRID = N // BLK
R = 400
C = 250


def _cos_kernel(et_ref, col_ref, cos_ref):
    e = et_ref[...]
    qn = e / jnp.sqrt(jnp.sum(e * e) + 1e-12)
    x = col_ref[...]
    ss = jnp.dot(x * x, jnp.ones((D, 1), jnp.float32),
                 preferred_element_type=jnp.float32) + 1e-12
    bn = x / jnp.sqrt(ss)
    cos_ref[...] = jnp.dot(bn, qn, preferred_element_type=jnp.float32)


def _vote_kernel(cos_ref, lab_ref, pred_ref, conf_ref, nconf_ref):
    cur = cos_ref[...]
    labs = lab_ref[...]
    row = jax.lax.broadcasted_iota(jnp.int32, (R, C), 0)
    col = jax.lax.broadcasted_iota(jnp.int32, (R, C), 1)
    flat = row * C + col
    big_i = jnp.int32(2**31 - 1)
    neg = jnp.float32(-jnp.inf)
    vals = []
    lbls = []
    for _ in range(10):
        m = jnp.max(cur)
        pos = jnp.min(jnp.where(cur == m, flat, big_i))
        sel = flat == pos
        vals.append(m)
        lbls.append(jnp.sum(jnp.where(sel, labs, 0)))
        cur = jnp.where(sel, neg, cur)
    nb_l = lbls[1:10]
    nb_v = vals[1:10]
    cnts = []
    for j in range(9):
        cj = jnp.int32(0)
        for k in range(9):
            cj = cj + (nb_l[j] == nb_l[k]).astype(jnp.int32)
        cnts.append(cj)
    best = cnts[0]
    for j in range(1, 9):
        best = jnp.maximum(best, cnts[j])
    winner = big_i
    for j in range(9):
        winner = jnp.minimum(winner, jnp.where(cnts[j] == best, nb_l[j], big_i))
    firstj = big_i
    for j in range(9):
        firstj = jnp.minimum(firstj, jnp.where(nb_l[j] == winner,
                                               jnp.int32(j), big_i))
    conf = jnp.float32(0.0)
    for j in range(9):
        conf = conf + jnp.where(firstj == j, nb_v[j], jnp.float32(0.0))
    pred_ref[...] = winner[None, None]
    conf_ref[...] = conf[None, None]
    nconf_ref[...] = (best.astype(jnp.float32) / jnp.float32(9.0))[None, None]


def kernel(embedding, embedding_collection, labels_int):
    et = embedding.reshape(D, 1)
    cos = pl.pallas_call(
        _cos_kernel,
        grid=(GRID,),
        in_specs=[
            pl.BlockSpec((D, 1), lambda i: (0, 0)),
            pl.BlockSpec((BLK, D), lambda i: (i, 0)),
        ],
        out_specs=pl.BlockSpec((BLK, 1), lambda i: (i, 0)),
        out_shape=jax.ShapeDtypeStruct((N, 1), jnp.float32),
    )(et, embedding_collection)
    pred, conf, nconf = pl.pallas_call(
        _vote_kernel,
        in_specs=[
            pl.BlockSpec((R, C), lambda: (0, 0)),
            pl.BlockSpec((R, C), lambda: (0, 0)),
        ],
        out_specs=[
            pl.BlockSpec((1, 1), lambda: (0, 0)),
            pl.BlockSpec((1, 1), lambda: (0, 0)),
            pl.BlockSpec((1, 1), lambda: (0, 0)),
        ],
        out_shape=[
            jax.ShapeDtypeStruct((1, 1), jnp.int32),
            jax.ShapeDtypeStruct((1, 1), jnp.float32),
            jax.ShapeDtypeStruct((1, 1), jnp.float32),
        ],
    )(cos.reshape(R, C), labels_int.reshape(R, C))
    return (pred[0, 0], conf[0, 0], nconf[0, 0])


# speedup vs baseline: 5.1088x; 2.0714x over previous
import jax
import jax.numpy as jnp
from jax import lax
from jax.experimental import pallas as pl

N = 100000
D = 128
BLK = 4000
GRID = N // BLK
R = GRID
C = BLK

_NT = (((1,), (1,)), ((), ()))


def _cos_kernel(e_ref, col_ref, cos_ref):
    e = e_ref[...]
    qn = e / jnp.sqrt(jnp.sum(e * e) + 1e-12)
    x = col_ref[...]
    ones = jnp.ones((1, D), jnp.float32)
    ss = lax.dot_general(ones, x * x, _NT,
                         preferred_element_type=jnp.float32)
    dp = lax.dot_general(qn, x, _NT,
                         preferred_element_type=jnp.float32)
    cos_ref[...] = (dp / jnp.sqrt(ss + 1e-12))[None]


def _vote_kernel(cos_ref, lab_ref, pred_ref, conf_ref, nconf_ref):
    cur = cos_ref[...]
    labs = lab_ref[...]
    row = jax.lax.broadcasted_iota(jnp.int32, (R, C), 0)
    col = jax.lax.broadcasted_iota(jnp.int32, (R, C), 1)
    flat = row * C + col
    big_i = jnp.int32(2**31 - 1)
    neg = jnp.float32(-jnp.inf)
    vals = []
    lbls = []
    for _ in range(10):
        m = jnp.max(cur)
        pos = jnp.min(jnp.where(cur == m, flat, big_i))
        sel = flat == pos
        vals.append(m)
        lbls.append(jnp.sum(jnp.where(sel, labs, 0)))
        cur = jnp.where(sel, neg, cur)
    nb_l = lbls[1:10]
    nb_v = vals[1:10]
    cnts = []
    for j in range(9):
        cj = jnp.int32(0)
        for k in range(9):
            cj = cj + (nb_l[j] == nb_l[k]).astype(jnp.int32)
        cnts.append(cj)
    best = cnts[0]
    for j in range(1, 9):
        best = jnp.maximum(best, cnts[j])
    winner = big_i
    for j in range(9):
        winner = jnp.minimum(winner, jnp.where(cnts[j] == best, nb_l[j], big_i))
    firstj = big_i
    for j in range(9):
        firstj = jnp.minimum(firstj, jnp.where(nb_l[j] == winner,
                                               jnp.int32(j), big_i))
    conf = jnp.float32(0.0)
    for j in range(9):
        conf = conf + jnp.where(firstj == j, nb_v[j], jnp.float32(0.0))
    pred_ref[...] = winner[None, None]
    conf_ref[...] = conf[None, None]
    nconf_ref[...] = (best.astype(jnp.float32) / jnp.float32(9.0))[None, None]


def kernel(embedding, embedding_collection, labels_int):
    cos = pl.pallas_call(
        _cos_kernel,
        grid=(GRID,),
        in_specs=[
            pl.BlockSpec((1, D), lambda i: (0, 0)),
            pl.BlockSpec((BLK, D), lambda i: (i, 0)),
        ],
        out_specs=pl.BlockSpec((1, 1, BLK), lambda i: (i, 0, 0)),
        out_shape=jax.ShapeDtypeStruct((GRID, 1, BLK), jnp.float32),
    )(embedding, embedding_collection)
    pred, conf, nconf = pl.pallas_call(
        _vote_kernel,
        in_specs=[
            pl.BlockSpec((R, C), lambda: (0, 0)),
            pl.BlockSpec((R, C), lambda: (0, 0)),
        ],
        out_specs=[
            pl.BlockSpec((1, 1), lambda: (0, 0)),
            pl.BlockSpec((1, 1), lambda: (0, 0)),
            pl.BlockSpec((1, 1), lambda: (0, 0)),
        ],
        out_shape=[
            jax.ShapeDtypeStruct((1, 1), jnp.int32),
            jax.ShapeDtypeStruct((1, 1), jnp.float32),
            jax.ShapeDtypeStruct((1, 1), jnp.float32),
        ],
    )(cos.reshape(R, C), labels_int.reshape(R, C))
    return (pred[0, 0], conf[0, 0], nconf[0, 0])


# speedup vs baseline: 5.5422x; 1.0848x over previous
import jax
import jax.numpy as jnp
from jax import lax
from jax.experimental import pallas as pl

N = 100000
D = 128
BLK = 5000
GRID = N // BLK
R = GRID
C = BLK

_NT = (((1,), (1,)), ((), ()))


def _cos_kernel(e_ref, col_ref, cos_ref):
    e = e_ref[...]
    qn = e / jnp.sqrt(jnp.sum(e * e) + 1e-12)
    x = col_ref[...]
    ones = jnp.ones((1, D), jnp.float32)
    ss = lax.dot_general(ones, x * x, _NT,
                         preferred_element_type=jnp.float32)
    dp = lax.dot_general(qn, x, _NT,
                         preferred_element_type=jnp.float32)
    cos_ref[...] = (dp / jnp.sqrt(ss + 1e-12))[None]


def _vote_kernel(cos_ref, lab_ref, pred_ref, conf_ref, nconf_ref):
    cur = cos_ref[...]
    labs = lab_ref[...]
    row = jax.lax.broadcasted_iota(jnp.int32, (R, C), 0)
    col = jax.lax.broadcasted_iota(jnp.int32, (R, C), 1)
    flat = row * C + col
    big_i = jnp.int32(2**31 - 1)
    neg = jnp.float32(-jnp.inf)
    vals = []
    lbls = []
    for _ in range(10):
        m = jnp.max(cur)
        pos = jnp.min(jnp.where(cur == m, flat, big_i))
        sel = flat == pos
        vals.append(m)
        lbls.append(jnp.sum(jnp.where(sel, labs, 0)))
        cur = jnp.where(sel, neg, cur)
    nb_l = lbls[1:10]
    nb_v = vals[1:10]
    cnts = []
    for j in range(9):
        cj = jnp.int32(0)
        for k in range(9):
            cj = cj + (nb_l[j] == nb_l[k]).astype(jnp.int32)
        cnts.append(cj)
    best = cnts[0]
    for j in range(1, 9):
        best = jnp.maximum(best, cnts[j])
    winner = big_i
    for j in range(9):
        winner = jnp.minimum(winner, jnp.where(cnts[j] == best, nb_l[j], big_i))
    firstj = big_i
    for j in range(9):
        firstj = jnp.minimum(firstj, jnp.where(nb_l[j] == winner,
                                               jnp.int32(j), big_i))
    conf = jnp.float32(0.0)
    for j in range(9):
        conf = conf + jnp.where(firstj == j, nb_v[j], jnp.float32(0.0))
    pred_ref[...] = winner[None, None]
    conf_ref[...] = conf[None, None]
    nconf_ref[...] = (best.astype(jnp.float32) / jnp.float32(9.0))[None, None]


def kernel(embedding, embedding_collection, labels_int):
    cos = pl.pallas_call(
        _cos_kernel,
        grid=(GRID,),
        in_specs=[
            pl.BlockSpec((1, D), lambda i: (0, 0)),
            pl.BlockSpec((BLK, D), lambda i: (i, 0)),
        ],
        out_specs=pl.BlockSpec((1, 1, BLK), lambda i: (i, 0, 0)),
        out_shape=jax.ShapeDtypeStruct((GRID, 1, BLK), jnp.float32),
    )(embedding, embedding_collection)
    pred, conf, nconf = pl.pallas_call(
        _vote_kernel,
        in_specs=[
            pl.BlockSpec((R, C), lambda: (0, 0)),
            pl.BlockSpec((R, C), lambda: (0, 0)),
        ],
        out_specs=[
            pl.BlockSpec((1, 1), lambda: (0, 0)),
            pl.BlockSpec((1, 1), lambda: (0, 0)),
            pl.BlockSpec((1, 1), lambda: (0, 0)),
        ],
        out_shape=[
            jax.ShapeDtypeStruct((1, 1), jnp.int32),
            jax.ShapeDtypeStruct((1, 1), jnp.float32),
            jax.ShapeDtypeStruct((1, 1), jnp.float32),
        ],
    )(cos.reshape(R, C), labels_int.reshape(R, C))
    return (pred[0, 0], conf[0, 0], nconf[0, 0])


# speedup vs baseline: 6.3291x; 1.1420x over previous
import jax
import jax.numpy as jnp
from jax import lax
from jax.experimental import pallas as pl

N = 100000
D = 128
BLK = 10000
GRID = N // BLK
R = GRID
C = BLK

_NT = (((1,), (1,)), ((), ()))


def _cos_kernel(e_ref, col_ref, cos_ref):
    e = e_ref[...]
    qn = e / jnp.sqrt(jnp.sum(e * e) + 1e-12)
    x = col_ref[...]
    ones = jnp.ones((1, D), jnp.float32)
    ss = lax.dot_general(ones, x * x, _NT,
                         preferred_element_type=jnp.float32)
    dp = lax.dot_general(qn, x, _NT,
                         preferred_element_type=jnp.float32)
    cos_ref[...] = (dp / jnp.sqrt(ss + 1e-12))[None]


def _vote_kernel(cos_ref, lab_ref, pred_ref, conf_ref, nconf_ref):
    cur = cos_ref[...]
    labs = lab_ref[...]
    row = jax.lax.broadcasted_iota(jnp.int32, (R, C), 0)
    col = jax.lax.broadcasted_iota(jnp.int32, (R, C), 1)
    flat = row * C + col
    big_i = jnp.int32(2**31 - 1)
    neg = jnp.float32(-jnp.inf)
    vals = []
    lbls = []
    for _ in range(10):
        m = jnp.max(cur)
        pos = jnp.min(jnp.where(cur == m, flat, big_i))
        sel = flat == pos
        vals.append(m)
        lbls.append(jnp.sum(jnp.where(sel, labs, 0)))
        cur = jnp.where(sel, neg, cur)
    nb_l = lbls[1:10]
    nb_v = vals[1:10]
    cnts = []
    for j in range(9):
        cj = jnp.int32(0)
        for k in range(9):
            cj = cj + (nb_l[j] == nb_l[k]).astype(jnp.int32)
        cnts.append(cj)
    best = cnts[0]
    for j in range(1, 9):
        best = jnp.maximum(best, cnts[j])
    winner = big_i
    for j in range(9):
        winner = jnp.minimum(winner, jnp.where(cnts[j] == best, nb_l[j], big_i))
    firstj = big_i
    for j in range(9):
        firstj = jnp.minimum(firstj, jnp.where(nb_l[j] == winner,
                                               jnp.int32(j), big_i))
    conf = jnp.float32(0.0)
    for j in range(9):
        conf = conf + jnp.where(firstj == j, nb_v[j], jnp.float32(0.0))
    pred_ref[...] = winner[None, None]
    conf_ref[...] = conf[None, None]
    nconf_ref[...] = (best.astype(jnp.float32) / jnp.float32(9.0))[None, None]


def kernel(embedding, embedding_collection, labels_int):
    cos = pl.pallas_call(
        _cos_kernel,
        grid=(GRID,),
        in_specs=[
            pl.BlockSpec((1, D), lambda i: (0, 0)),
            pl.BlockSpec((BLK, D), lambda i: (i, 0)),
        ],
        out_specs=pl.BlockSpec((1, 1, BLK), lambda i: (i, 0, 0)),
        out_shape=jax.ShapeDtypeStruct((GRID, 1, BLK), jnp.float32),
    )(embedding, embedding_collection)
    pred, conf, nconf = pl.pallas_call(
        _vote_kernel,
        in_specs=[
            pl.BlockSpec((R, C), lambda: (0, 0)),
            pl.BlockSpec((R, C), lambda: (0, 0)),
        ],
        out_specs=[
            pl.BlockSpec((1, 1), lambda: (0, 0)),
            pl.BlockSpec((1, 1), lambda: (0, 0)),
            pl.BlockSpec((1, 1), lambda: (0, 0)),
        ],
        out_shape=[
            jax.ShapeDtypeStruct((1, 1), jnp.int32),
            jax.ShapeDtypeStruct((1, 1), jnp.float32),
            jax.ShapeDtypeStruct((1, 1), jnp.float32),
        ],
    )(cos.reshape(R, C), labels_int.reshape(R, C))
    return (pred[0, 0], conf[0, 0], nconf[0, 0])


# speedup vs baseline: 6.4616x; 1.0209x over previous
import jax
import jax.numpy as jnp
from jax import lax
from jax.experimental import pallas as pl

N = 100000
D = 128
BLK = 20000
GRID = N // BLK
R = GRID
C = BLK

_NT = (((1,), (1,)), ((), ()))


def _cos_kernel(e_ref, col_ref, cos_ref):
    e = e_ref[...]
    qn = e / jnp.sqrt(jnp.sum(e * e) + 1e-12)
    x = col_ref[...]
    ones = jnp.ones((1, D), jnp.float32)
    ss = lax.dot_general(ones, x * x, _NT,
                         preferred_element_type=jnp.float32)
    dp = lax.dot_general(qn, x, _NT,
                         preferred_element_type=jnp.float32)
    cos_ref[...] = (dp / jnp.sqrt(ss + 1e-12))[None]


def _vote_kernel(cos_ref, lab_ref, pred_ref, conf_ref, nconf_ref):
    cur = cos_ref[...]
    labs = lab_ref[...]
    row = jax.lax.broadcasted_iota(jnp.int32, (R, C), 0)
    col = jax.lax.broadcasted_iota(jnp.int32, (R, C), 1)
    flat = row * C + col
    big_i = jnp.int32(2**31 - 1)
    neg = jnp.float32(-jnp.inf)
    vals = []
    lbls = []
    for _ in range(10):
        m = jnp.max(cur)
        pos = jnp.min(jnp.where(cur == m, flat, big_i))
        sel = flat == pos
        vals.append(m)
        lbls.append(jnp.sum(jnp.where(sel, labs, 0)))
        cur = jnp.where(sel, neg, cur)
    nb_l = lbls[1:10]
    nb_v = vals[1:10]
    cnts = []
    for j in range(9):
        cj = jnp.int32(0)
        for k in range(9):
            cj = cj + (nb_l[j] == nb_l[k]).astype(jnp.int32)
        cnts.append(cj)
    best = cnts[0]
    for j in range(1, 9):
        best = jnp.maximum(best, cnts[j])
    winner = big_i
    for j in range(9):
        winner = jnp.minimum(winner, jnp.where(cnts[j] == best, nb_l[j], big_i))
    firstj = big_i
    for j in range(9):
        firstj = jnp.minimum(firstj, jnp.where(nb_l[j] == winner,
                                               jnp.int32(j), big_i))
    conf = jnp.float32(0.0)
    for j in range(9):
        conf = conf + jnp.where(firstj == j, nb_v[j], jnp.float32(0.0))
    pred_ref[...] = winner[None, None]
    conf_ref[...] = conf[None, None]
    nconf_ref[...] = (best.astype(jnp.float32) / jnp.float32(9.0))[None, None]


def kernel(embedding, embedding_collection, labels_int):
    cos = pl.pallas_call(
        _cos_kernel,
        grid=(GRID,),
        in_specs=[
            pl.BlockSpec((1, D), lambda i: (0, 0)),
            pl.BlockSpec((BLK, D), lambda i: (i, 0)),
        ],
        out_specs=pl.BlockSpec((1, 1, BLK), lambda i: (i, 0, 0)),
        out_shape=jax.ShapeDtypeStruct((GRID, 1, BLK), jnp.float32),
    )(embedding, embedding_collection)
    pred, conf, nconf = pl.pallas_call(
        _vote_kernel,
        in_specs=[
            pl.BlockSpec((R, C), lambda: (0, 0)),
            pl.BlockSpec((R, C), lambda: (0, 0)),
        ],
        out_specs=[
            pl.BlockSpec((1, 1), lambda: (0, 0)),
            pl.BlockSpec((1, 1), lambda: (0, 0)),
            pl.BlockSpec((1, 1), lambda: (0, 0)),
        ],
        out_shape=[
            jax.ShapeDtypeStruct((1, 1), jnp.int32),
            jax.ShapeDtypeStruct((1, 1), jnp.float32),
            jax.ShapeDtypeStruct((1, 1), jnp.float32),
        ],
    )(cos.reshape(R, C), labels_int.reshape(R, C))
    return (pred[0, 0], conf[0, 0], nconf[0, 0])
